# baseline (speedup 1.0000x reference)
import jax
import jax.numpy as jnp
from jax.experimental import pallas as pl
from jax.experimental.pallas import tpu as pltpu

TM1 = 128
TM2 = 2048


def _merge_body(w_ref, ri_ref, s_ref, a_ref, feature_ref, enc_ref, enc_t_ref,
                gc1_w_ref, b1_ref, gc2_w_ref, b2_ref, sw_ref,
                fin_ref, u1_ref, u4_ref, h1_scr):
    i = pl.program_id(0)

    @pl.when(i == 0)
    def _():
        h1 = jnp.dot(feature_ref[...], gc1_w_ref[...],
                     preferred_element_type=jnp.float32)
        h1_scr[...] = h1.astype(jnp.bfloat16)
        enc = enc_ref[...]
        enc_t = enc_t_ref[...]
        ew = enc * sw_ref[...]
        t1 = jnp.dot(enc_t, h1, preferred_element_type=jnp.float32)
        u3 = jnp.dot(ew, t1, preferred_element_type=jnp.float32) + b1_ref[...]
        g3 = jnp.dot(u3, gc2_w_ref[...], preferred_element_type=jnp.float32)
        t2 = jnp.dot(enc_t, g3, preferred_element_type=jnp.float32)
        u4_ref[...] = jnp.dot(ew, t2,
                              preferred_element_type=jnp.float32) + b2_ref[...]

    a = a_ref[...]
    bbp = w_ref[0, 0] * a[0]
    for r in range(1, a.shape[0]):
        bbp = bbp + w_ref[r, 0] * a[r]
    a0, a1, a2 = a[0], a[1], a[2]
    p0 = jnp.where(a0 > 0, 0.6 * a0 + 0.4, 0.0)
    p1 = jnp.where(a1 > 0, 0.6 * a1 + 0.4, 0.0)
    p2 = jnp.where(a2 > 0, 0.6 * a2 + 0.4, 0.0)
    e = (a0 * (ri_ref[1, 0] * p1 + ri_ref[2, 0] * p2)
         + a1 * (ri_ref[0, 1] * p0 + ri_ref[2, 1] * p2)
         + a2 * (ri_ref[0, 2] * p0 + ri_ref[1, 2] * p1))
    fin = (bbp + s_ref[0] * jnp.tanh(e)).astype(jnp.bfloat16)
    fin_ref[...] = fin
    u1_ref[...] = jnp.dot(fin, h1_scr[...],
                          preferred_element_type=jnp.float32) + b1_ref[...]


def _spmm2_body(fin_ref, u1b_ref, u1i_ref, u4_ref, w2_ref, b2_ref,
                res_ref, br1_ref, br2_ref):
    v = jnp.dot(fin_ref[...], u1b_ref[...].astype(jnp.bfloat16),
                preferred_element_type=jnp.float32)
    u2 = jnp.dot(v, w2_ref[...], preferred_element_type=jnp.float32) + b2_ref[...]
    u1i = u1i_ref[...]
    u4 = u4_ref[...]
    s = (u1i + u2) * 0.5
    r = (s + u4) * 0.5

    def nrm(x):
        n = jnp.sqrt(jnp.sum(x * x, axis=1, keepdims=True))
        return x / jnp.maximum(n, 1e-12)

    res_ref[...] = nrm(r)
    br1_ref[...] = nrm(s)
    br2_ref[...] = nrm(u4)


def kernel(feature, A, encode, gc1_w, gc1_b, gc2_w, gc2_b, weight_b,
           relation_interaction, interaction_strength, struct_weight):
    n, nfeat = feature.shape
    out = gc1_w.shape[1]
    nrel = A.shape[0]
    enc_t = encode.T
    sw = struct_weight.reshape(1, -1)
    b1 = gc1_b.reshape(1, -1)
    b2 = gc2_b.reshape(1, -1)

    smem = pl.BlockSpec(memory_space=pltpu.SMEM)
    const2d = lambda bs: pl.BlockSpec(bs, lambda i: (0, 0))
    fin, u1, u4 = pl.pallas_call(
        _merge_body,
        grid=(n // TM1,),
        in_specs=[
            smem,
            smem,
            smem,
            pl.BlockSpec((nrel, TM1, n), lambda i: (0, i, 0)),
            const2d((n, nfeat)),
            const2d((n, nrel)),
            const2d((nrel, n)),
            const2d((nfeat, out)),
            const2d((1, out)),
            const2d((out, out)),
            const2d((1, out)),
            const2d((1, nrel)),
        ],
        out_specs=[
            pl.BlockSpec((TM1, n), lambda i: (i, 0)),
            pl.BlockSpec((TM1, out), lambda i: (i, 0)),
            pl.BlockSpec((n, out), lambda i: (0, 0)),
        ],
        out_shape=[jax.ShapeDtypeStruct((n, n), jnp.bfloat16),
                   jax.ShapeDtypeStruct((n, out), jnp.float32),
                   jax.ShapeDtypeStruct((n, out), jnp.float32)],
        scratch_shapes=[pltpu.VMEM((n, out), jnp.bfloat16)],
    )(weight_b, relation_interaction, interaction_strength, A,
      feature, encode, enc_t, gc1_w, b1, gc2_w, b2, sw)

    res, br1, br2 = pl.pallas_call(
        _spmm2_body,
        grid=(n // TM2,),
        in_specs=[
            pl.BlockSpec((TM2, n), lambda i: (i, 0)),
            pl.BlockSpec((n, out), lambda i: (0, 0)),
            pl.BlockSpec((TM2, out), lambda i: (i, 0)),
            pl.BlockSpec((TM2, out), lambda i: (i, 0)),
            pl.BlockSpec((out, out), lambda i: (0, 0)),
            pl.BlockSpec((1, out), lambda i: (0, 0)),
        ],
        out_specs=[
            pl.BlockSpec((TM2, out), lambda i: (i, 0)),
            pl.BlockSpec((TM2, out), lambda i: (i, 0)),
            pl.BlockSpec((TM2, out), lambda i: (i, 0)),
        ],
        out_shape=[jax.ShapeDtypeStruct((n, out), jnp.float32),
                   jax.ShapeDtypeStruct((n, out), jnp.float32),
                   jax.ShapeDtypeStruct((n, out), jnp.float32)],
    )(fin, u1, u1, u4, gc2_w, b2)

    return res, br1, br2


# speedup vs baseline: 1.0214x; 1.0214x over previous
import jax
import jax.numpy as jnp
from jax.experimental import pallas as pl
from jax.experimental.pallas import tpu as pltpu

TM1 = 128
TM2 = 1024


def _merge_body(w_ref, ri_ref, s_ref, a_ref, feature_ref, enc_ref, enc_t_ref,
                gc1_w_ref, b1_ref, gc2_w_ref, b2_ref, sw_ref,
                fin_ref, u1_ref, u4_ref, h1_scr):
    i = pl.program_id(0)

    @pl.when(i == 0)
    def _():
        h1 = jnp.dot(feature_ref[...], gc1_w_ref[...],
                     preferred_element_type=jnp.float32)
        h1_scr[...] = h1.astype(jnp.bfloat16)
        enc = enc_ref[...]
        enc_t = enc_t_ref[...]
        ew = enc * sw_ref[...]
        t1 = jnp.dot(enc_t, h1, preferred_element_type=jnp.float32)
        u3 = jnp.dot(ew, t1, preferred_element_type=jnp.float32) + b1_ref[...]
        g3 = jnp.dot(u3, gc2_w_ref[...], preferred_element_type=jnp.float32)
        t2 = jnp.dot(enc_t, g3, preferred_element_type=jnp.float32)
        u4_ref[...] = jnp.dot(ew, t2,
                              preferred_element_type=jnp.float32) + b2_ref[...]

    a = a_ref[...]
    bbp = w_ref[0, 0] * a[0]
    for r in range(1, a.shape[0]):
        bbp = bbp + w_ref[r, 0] * a[r]
    a0, a1, a2 = a[0], a[1], a[2]
    p0 = jnp.where(a0 > 0, 0.6 * a0 + 0.4, 0.0)
    p1 = jnp.where(a1 > 0, 0.6 * a1 + 0.4, 0.0)
    p2 = jnp.where(a2 > 0, 0.6 * a2 + 0.4, 0.0)
    e = (a0 * (ri_ref[1, 0] * p1 + ri_ref[2, 0] * p2)
         + a1 * (ri_ref[0, 1] * p0 + ri_ref[2, 1] * p2)
         + a2 * (ri_ref[0, 2] * p0 + ri_ref[1, 2] * p1))
    fin = (bbp + s_ref[0] * jnp.tanh(e)).astype(jnp.bfloat16)
    fin_ref[...] = fin
    u1_ref[...] = jnp.dot(fin, h1_scr[...],
                          preferred_element_type=jnp.float32) + b1_ref[...]


def _spmm2_body(fin_ref, u1b_ref, u1i_ref, u4_ref, w2_ref, b2_ref,
                res_ref, br1_ref, br2_ref):
    v = jnp.dot(fin_ref[...], u1b_ref[...].astype(jnp.bfloat16),
                preferred_element_type=jnp.float32)
    u2 = jnp.dot(v, w2_ref[...], preferred_element_type=jnp.float32) + b2_ref[...]
    u1i = u1i_ref[...]
    u4 = u4_ref[...]
    s = (u1i + u2) * 0.5
    r = (s + u4) * 0.5

    def nrm(x):
        n = jnp.sqrt(jnp.sum(x * x, axis=1, keepdims=True))
        return x / jnp.maximum(n, 1e-12)

    res_ref[...] = nrm(r)
    br1_ref[...] = nrm(s)
    br2_ref[...] = nrm(u4)


def kernel(feature, A, encode, gc1_w, gc1_b, gc2_w, gc2_b, weight_b,
           relation_interaction, interaction_strength, struct_weight):
    n, nfeat = feature.shape
    out = gc1_w.shape[1]
    nrel = A.shape[0]
    enc_t = encode.T
    sw = struct_weight.reshape(1, -1)
    b1 = gc1_b.reshape(1, -1)
    b2 = gc2_b.reshape(1, -1)

    smem = pl.BlockSpec(memory_space=pltpu.SMEM)
    const2d = lambda bs: pl.BlockSpec(bs, lambda i: (0, 0))
    fin, u1, u4 = pl.pallas_call(
        _merge_body,
        grid=(n // TM1,),
        in_specs=[
            smem,
            smem,
            smem,
            pl.BlockSpec((nrel, TM1, n), lambda i: (0, i, 0)),
            const2d((n, nfeat)),
            const2d((n, nrel)),
            const2d((nrel, n)),
            const2d((nfeat, out)),
            const2d((1, out)),
            const2d((out, out)),
            const2d((1, out)),
            const2d((1, nrel)),
        ],
        out_specs=[
            pl.BlockSpec((TM1, n), lambda i: (i, 0)),
            pl.BlockSpec((TM1, out), lambda i: (i, 0)),
            pl.BlockSpec((n, out), lambda i: (0, 0)),
        ],
        out_shape=[jax.ShapeDtypeStruct((n, n), jnp.bfloat16),
                   jax.ShapeDtypeStruct((n, out), jnp.float32),
                   jax.ShapeDtypeStruct((n, out), jnp.float32)],
        scratch_shapes=[pltpu.VMEM((n, out), jnp.bfloat16)],
    )(weight_b, relation_interaction, interaction_strength, A,
      feature, encode, enc_t, gc1_w, b1, gc2_w, b2, sw)

    res, br1, br2 = pl.pallas_call(
        _spmm2_body,
        grid=(n // TM2,),
        in_specs=[
            pl.BlockSpec((TM2, n), lambda i: (i, 0)),
            pl.BlockSpec((n, out), lambda i: (0, 0)),
            pl.BlockSpec((TM2, out), lambda i: (i, 0)),
            pl.BlockSpec((TM2, out), lambda i: (i, 0)),
            pl.BlockSpec((out, out), lambda i: (0, 0)),
            pl.BlockSpec((1, out), lambda i: (0, 0)),
        ],
        out_specs=[
            pl.BlockSpec((TM2, out), lambda i: (i, 0)),
            pl.BlockSpec((TM2, out), lambda i: (i, 0)),
            pl.BlockSpec((TM2, out), lambda i: (i, 0)),
        ],
        out_shape=[jax.ShapeDtypeStruct((n, out), jnp.float32),
                   jax.ShapeDtypeStruct((n, out), jnp.float32),
                   jax.ShapeDtypeStruct((n, out), jnp.float32)],
    )(fin, u1, u1, u4, gc2_w, b2)

    return res, br1, br2
